# baseline (speedup 1.0000x reference)
import functools

import jax
import jax.numpy as jnp
import numpy as np
from jax import lax
from jax.experimental import pallas as pl
from jax.experimental.pallas import tpu as pltpu
from jax.experimental.pallas import tpu_sc as plsc


_SC_NW = 32
_SC_CH = 128


def _sc_gather(table, idx):
    b = idx.shape[0]
    d = table.shape[1]
    b_per_w = b // _SC_NW
    nch = b_per_w // _SC_CH
    mesh = plsc.VectorSubcoreMesh(core_axis_name="c", subcore_axis_name="s")

    @functools.partial(
        pl.kernel, mesh=mesh,
        out_type=jax.ShapeDtypeStruct((b, d), jnp.float32),
        scratch_types=[
            pltpu.VMEM((_SC_CH,), jnp.int32),
            pltpu.VMEM((_SC_CH, d), jnp.float32),
            pltpu.SemaphoreType.DMA,
        ],
    )
    def k(table_hbm, idx_hbm, out_hbm, idx_v, rows_v, sem):
        wid = lax.axis_index("s") * 2 + lax.axis_index("c")
        base = wid * b_per_w

        def body(i, carry):
            off = base + i * _SC_CH
            pltpu.sync_copy(idx_hbm.at[pl.ds(off, _SC_CH)], idx_v)
            pltpu.async_copy(table_hbm.at[idx_v], rows_v, sem).wait()
            pltpu.sync_copy(rows_v, out_hbm.at[pl.ds(off, _SC_CH)])
            return carry

        lax.fori_loop(0, nch, body, 0)

    return k(table, idx)


def _sc_scatter_add(vals0, vals1, dstv, npad):
    epad = dstv.shape[0]
    e_per_s = epad // 16
    nch = e_per_s // _SC_CH
    stripe = npad // 16
    mesh = plsc.VectorSubcoreMesh(core_axis_name="c", subcore_axis_name="s")

    @functools.partial(
        pl.kernel, mesh=mesh,
        out_type=(jax.ShapeDtypeStruct((npad, 128), jnp.float32),
                  jax.ShapeDtypeStruct((npad, 128), jnp.float32)),
        scratch_types=[
            pltpu.VMEM_SHARED((npad, 128), jnp.float32),
            pltpu.VMEM((_SC_CH,), jnp.int32),
            pltpu.VMEM((_SC_CH, 128), jnp.float32),
            pltpu.SemaphoreType.DMA,
        ],
    )
    def k(v0_hbm, v1_hbm, zero_hbm, dst_hbm, o0_hbm, o1_hbm,
          acc, idx_v, rows_v, sem):
        cid = lax.axis_index("c")
        sid = lax.axis_index("s")

        @pl.when(sid == 0)
        def _():
            pltpu.sync_copy(zero_hbm, acc)

        plsc.subcore_barrier()

        def make_body(v_hbm):
            def body(i, carry):
                off = sid * e_per_s + i * _SC_CH
                pltpu.sync_copy(dst_hbm.at[pl.ds(off, _SC_CH)], idx_v)
                pltpu.async_copy(v_hbm.at[pl.ds(off, _SC_CH)], rows_v,
                                 sem).wait()
                pltpu.sync_copy(rows_v, acc.at[idx_v], add=True)
                return carry
            return body

        @pl.when(cid == 0)
        def _():
            lax.fori_loop(0, nch, make_body(v0_hbm), 0)

        @pl.when(cid == 1)
        def _():
            lax.fori_loop(0, nch, make_body(v1_hbm), 0)

        plsc.subcore_barrier()

        @pl.when(cid == 0)
        def _():
            pltpu.sync_copy(acc.at[pl.ds(sid * stripe, stripe)],
                            o0_hbm.at[pl.ds(sid * stripe, stripe)])

        @pl.when(cid == 1)
        def _():
            pltpu.sync_copy(acc.at[pl.ds(sid * stripe, stripe)],
                            o1_hbm.at[pl.ds(sid * stripe, stripe)])

    zero = jnp.zeros((npad, 128), jnp.float32)
    o0, o1 = k(vals0, vals1, zero, dstv)
    return jnp.concatenate([o0, o1], axis=1)

K = 16
BLK = 256
CHUNK = 512
NEG_INF = float("-inf")


def _leaky(v):
    return jnp.where(v > 0, v, 0.01 * v)


def _top16_merge(run_vals, run_idx, vals, cbase):
    r = run_vals.shape[0]
    cand_v = jnp.concatenate([run_vals, vals], axis=1)
    ncand = cand_v.shape[1]
    pos2 = lax.broadcasted_iota(jnp.int32, (r, ncand), 1)
    pos16 = lax.broadcasted_iota(jnp.int32, (r, K), 1)
    new_v = []
    new_i = []
    for _ in range(K):
        m = jnp.max(cand_v, axis=1, keepdims=True)
        first = jnp.min(jnp.where(cand_v == m, pos2, ncand),
                        axis=1, keepdims=True)
        from_run = jnp.sum(jnp.where(pos16 == first, run_idx, 0),
                           axis=1, keepdims=True)
        picked_i = jnp.where(first < K, from_run, first - K + cbase)
        new_v.append(m)
        new_i.append(picked_i)
        cand_v = jnp.where(pos2 == first, NEG_INF, cand_v)
    return (jnp.concatenate(new_v, axis=1),
            jnp.concatenate(new_i, axis=1))


def _knn_body(clo_ref, cnt_ref, hp_ref, sqr_ref, sqc_ref, rlo_ref, rhi_ref,
              out_ref):
    b = pl.program_id(0)
    h_blk = hp_ref[pl.ds(b * BLK, BLK), :]
    sq_blk = sqr_ref[...]
    rlo = rlo_ref[...]
    rhi = rhi_ref[...]
    rowid = b * BLK + lax.broadcasted_iota(jnp.int32, (BLK, 1), 0)

    run_v0 = jnp.full((BLK, K), NEG_INF, jnp.float32)
    run_i0 = lax.broadcasted_iota(jnp.int32, (BLK, K), 1)

    clo = clo_ref[b]
    cnt = cnt_ref[b]

    def body(j, carry):
        run_v, run_i = carry
        ci = clo + j
        c = ci * CHUNK
        hc = hp_ref[pl.ds(c, CHUNK), :]
        sq_c = sqc_ref[pl.ds(ci, 1), :]
        dot = lax.dot_general(h_blk, hc, (((1,), (1,)), ((), ())),
                              preferred_element_type=jnp.float32)
        d2 = (sq_blk + sq_c) - 2.0 * dot
        colid = c + lax.broadcasted_iota(jnp.int32, (1, CHUNK), 1)
        valid = (colid >= rlo) & (colid < rhi) & (rowid != colid)
        neg = jnp.where(valid, -d2, NEG_INF)
        return _top16_merge(run_v, run_i, neg, c)

    run_v, run_i = lax.fori_loop(0, cnt, body, (run_v0, run_i0))
    out_ref[...] = jnp.pad(run_i, ((0, 0), (0, 128 - K)))


def _knn_pallas(hp, rlo, rhi, clo, cnt):
    npad = hp.shape[0]
    grid = npad // BLK
    f = hp.shape[1]
    nc = npad // CHUNK
    sqv = jnp.sum(hp * hp, axis=1)
    out = pl.pallas_call(
        _knn_body,
        grid_spec=pltpu.PrefetchScalarGridSpec(
            num_scalar_prefetch=2,
            grid=(grid,),
            in_specs=[
                pl.BlockSpec((npad, f), lambda b, *_: (0, 0)),
                pl.BlockSpec((BLK, 1), lambda b, *_: (b, 0)),
                pl.BlockSpec((nc, CHUNK), lambda b, *_: (0, 0)),
                pl.BlockSpec((BLK, 1), lambda b, *_: (b, 0)),
                pl.BlockSpec((BLK, 1), lambda b, *_: (b, 0)),
            ],
            out_specs=pl.BlockSpec((BLK, 128), lambda b, *_: (b, 0)),
        ),
        out_shape=jax.ShapeDtypeStruct((npad, 128), jnp.int32),
    )(clo, cnt, hp, sqv.reshape(npad, 1), sqv.reshape(nc, CHUNK), rlo, rhi)
    return out[:, :K]


def _conv_knn_body(h_ref, hj_ref, wa_ref, ba_ref, wb_ref, bb_ref, out_ref):
    h_blk = h_ref[...]
    hj = hj_ref[...]
    xi = jnp.repeat(h_blk, K, axis=0)
    m = jnp.concatenate([xi, hj - xi], axis=1)
    a1 = _leaky(jnp.dot(m, wa_ref[...], preferred_element_type=jnp.float32)
                + ba_ref[...][None, :])
    a2 = _leaky(jnp.dot(a1, wb_ref[...], preferred_element_type=jnp.float32)
                + bb_ref[...][None, :])
    odim = a2.shape[1]
    a3 = a2.reshape(BLK, K, odim)
    acc = a3[:, 0, :]
    for t in range(1, K):
        acc = acc + a3[:, t, :]
    out_ref[...] = acc


def _conv_knn(h, hj, Wa, ba, Wb, bb):
    npad, f = h.shape
    hdim = Wa.shape[1]
    odim = Wb.shape[1]
    grid = npad // BLK
    return pl.pallas_call(
        _conv_knn_body,
        grid=(grid,),
        in_specs=[
            pl.BlockSpec((BLK, f), lambda b: (b, 0)),
            pl.BlockSpec((BLK * K, f), lambda b: (b, 0)),
            pl.BlockSpec((2 * f, hdim), lambda b: (0, 0)),
            pl.BlockSpec((hdim,), lambda b: (0,)),
            pl.BlockSpec((hdim, odim), lambda b: (0, 0)),
            pl.BlockSpec((odim,), lambda b: (0,)),
        ],
        out_specs=pl.BlockSpec((BLK, odim), lambda b: (b, 0)),
        out_shape=jax.ShapeDtypeStruct((npad, odim), jnp.float32),
    )(h, hj, Wa, ba, Wb, bb)


EBLK = 4096


def _conv1_body(xi_ref, xj_ref, wa_ref, ba_ref, wb_ref, bb_ref, out_ref):
    xi = xi_ref[...]
    m = jnp.concatenate([xi, xj_ref[...] - xi], axis=1)
    a1 = _leaky(jnp.dot(m, wa_ref[...], preferred_element_type=jnp.float32)
                + ba_ref[...][None, :])
    out_ref[...] = _leaky(
        jnp.dot(a1, wb_ref[...], preferred_element_type=jnp.float32)
        + bb_ref[...][None, :])


def _conv1(xi, xj, Wa, ba, Wb, bb):
    epad, f = xi.shape
    hdim = Wa.shape[1]
    odim = Wb.shape[1]
    grid = epad // EBLK
    return pl.pallas_call(
        _conv1_body,
        grid=(grid,),
        in_specs=[
            pl.BlockSpec((EBLK, f), lambda b: (b, 0)),
            pl.BlockSpec((EBLK, f), lambda b: (b, 0)),
            pl.BlockSpec((2 * f, hdim), lambda b: (0, 0)),
            pl.BlockSpec((hdim,), lambda b: (0,)),
            pl.BlockSpec((hdim, odim), lambda b: (0, 0)),
            pl.BlockSpec((odim,), lambda b: (0,)),
        ],
        out_specs=pl.BlockSpec((EBLK, odim), lambda b: (b, 0)),
        out_shape=jax.ShapeDtypeStruct((epad, odim), jnp.float32),
    )(xi, xj, Wa, ba, Wb, bb)


def _head_body(x_ref, s1_ref, s2_ref, s3_ref, s4_ref,
               wpa_ref, bpa_ref, wpb_ref, bpb_ref, wh_ref, bh_ref, o_ref):
    d = x_ref.shape[1]
    acc = jnp.dot(x_ref[...], wpa_ref[:d, :],
                  preferred_element_type=jnp.float32)
    for t, s_ref in enumerate((s1_ref, s2_ref, s3_ref, s4_ref)):
        acc = acc + jnp.dot(s_ref[...], wpa_ref[d + 256 * t:d + 256 * (t + 1), :],
                            preferred_element_type=jnp.float32)
    h1 = _leaky(acc + bpa_ref[...][None, :])
    h2 = _leaky(jnp.dot(h1, wpb_ref[...], preferred_element_type=jnp.float32)
                + bpb_ref[...][None, :])
    o_ref[...] = jnp.dot(h2, wh_ref[...], preferred_element_type=jnp.float32) \
        + bh_ref[...][None, :]


def _head(xp, skips, Wpa, bpa, Wpb, bpb, Wh, bh):
    npad, d = xp.shape
    f = Wpa.shape[0]
    grid = npad // BLK
    out = pl.pallas_call(
        _head_body,
        grid=(grid,),
        in_specs=[
            pl.BlockSpec((BLK, d), lambda i: (i, 0)),
            pl.BlockSpec((BLK, 256), lambda i: (i, 0)),
            pl.BlockSpec((BLK, 256), lambda i: (i, 0)),
            pl.BlockSpec((BLK, 256), lambda i: (i, 0)),
            pl.BlockSpec((BLK, 256), lambda i: (i, 0)),
            pl.BlockSpec((f, 336), lambda i: (0, 0)),
            pl.BlockSpec((336,), lambda i: (0,)),
            pl.BlockSpec((336, 256), lambda i: (0, 0)),
            pl.BlockSpec((256,), lambda i: (0,)),
            pl.BlockSpec((256, 128), lambda i: (0, 0)),
            pl.BlockSpec((128,), lambda i: (0,)),
        ],
        out_specs=pl.BlockSpec((BLK, 128), lambda i: (i, 0)),
        out_shape=jax.ShapeDtypeStruct((npad, 128), jnp.float32),
    )(xp, skips[0], skips[1], skips[2], skips[3], Wpa, bpa, Wpb, bpb,
      jnp.zeros((256, 128), jnp.float32).at[:, :1].set(Wh),
      jnp.zeros((128,), jnp.float32).at[:1].set(bh))
    return out[:, 0]


def kernel(x, W1a, b1a, W1b, b1b, W2a, b2a, W2b, b2b, W3a, b3a, W3b, b3b,
           W4a, b4a, W4b, b4b, Wpa, bpa, Wpb, bpb, Wh, bh, edge_index, batch):
    n, d = x.shape
    npad = ((n + BLK - 1) // BLK) * BLK
    e = edge_index.shape[1]
    epad = ((e + EBLK - 1) // EBLK) * EBLK

    batchp = jnp.full((npad,), 127, jnp.int32).at[:n].set(batch)
    rlo = jnp.searchsorted(batchp, batchp, side="left").astype(jnp.int32)
    rhi = jnp.searchsorted(batchp, batchp, side="right").astype(jnp.int32)
    lo = rlo[::BLK]
    hi = rhi[BLK - 1:: BLK]
    clo = lo // CHUNK
    cnt = (hi + CHUNK - 1) // CHUNK - clo
    rlo = rlo.reshape(npad, 1)
    rhi = rhi.reshape(npad, 1)

    src0 = jnp.zeros((epad,), jnp.int32).at[:e].set(edge_index[0])
    dst0 = jnp.full((epad,), n, jnp.int32).at[:e].set(edge_index[1])
    xp = jnp.zeros((npad, d), jnp.float32).at[:n].set(x)
    xi = _sc_gather(xp, dst0)
    xj = _sc_gather(xp, src0)
    hdn = _conv1(xi, xj, W1a, b1a, W1b, b1b)
    h1 = jax.ops.segment_sum(hdn, dst0, num_segments=n)
    h1p = jnp.zeros((npad, 256), jnp.float32).at[:n].set(h1)

    hp = h1p
    skips = [h1p]
    for (Wa, ba, Wb, bb) in ((W2a, b2a, W2b, b2b), (W3a, b3a, W3b, b3b),
                             (W4a, b4a, W4b, b4b)):
        idx = _knn_pallas(hp, rlo, rhi, clo, cnt)
        hj = _sc_gather(hp, idx.reshape(-1))
        hp = _conv_knn(hp, hj, Wa, ba, Wb, bb)
        skips.append(hp)

    return _head(xp, skips, Wpa, bpa, Wpb, bpb, Wh, bh)[:n]


# speedup vs baseline: 1.0759x; 1.0759x over previous
import functools

import jax
import jax.numpy as jnp
import numpy as np
from jax import lax
from jax.experimental import pallas as pl
from jax.experimental.pallas import tpu as pltpu
from jax.experimental.pallas import tpu_sc as plsc


_SC_NW = 32
_SC_CH = 128


def _sc_gather(table, idx):
    b = idx.shape[0]
    d = table.shape[1]
    b_per_w = b // _SC_NW
    nch = b_per_w // _SC_CH
    mesh = plsc.VectorSubcoreMesh(core_axis_name="c", subcore_axis_name="s")

    @functools.partial(
        pl.kernel, mesh=mesh,
        out_type=jax.ShapeDtypeStruct((b, d), jnp.float32),
        scratch_types=[
            pltpu.VMEM((_SC_CH,), jnp.int32),
            pltpu.VMEM((_SC_CH, d), jnp.float32),
            pltpu.SemaphoreType.DMA,
        ],
    )
    def k(table_hbm, idx_hbm, out_hbm, idx_v, rows_v, sem):
        wid = lax.axis_index("s") * 2 + lax.axis_index("c")
        base = wid * b_per_w

        def body(i, carry):
            off = base + i * _SC_CH
            pltpu.sync_copy(idx_hbm.at[pl.ds(off, _SC_CH)], idx_v)
            pltpu.async_copy(table_hbm.at[idx_v], rows_v, sem).wait()
            pltpu.sync_copy(rows_v, out_hbm.at[pl.ds(off, _SC_CH)])
            return carry

        lax.fori_loop(0, nch, body, 0)

    return k(table, idx)


def _sc_scatter_add(vals0, vals1, dstv, npad):
    epad = dstv.shape[0]
    e_per_s = epad // 16
    nch = e_per_s // _SC_CH
    stripe = npad // 16
    mesh = plsc.VectorSubcoreMesh(core_axis_name="c", subcore_axis_name="s")

    @functools.partial(
        pl.kernel, mesh=mesh,
        out_type=(jax.ShapeDtypeStruct((npad, 128), jnp.float32),
                  jax.ShapeDtypeStruct((npad, 128), jnp.float32)),
        scratch_types=[
            pltpu.VMEM_SHARED((npad, 128), jnp.float32),
            pltpu.VMEM((_SC_CH,), jnp.int32),
            pltpu.VMEM((_SC_CH, 128), jnp.float32),
            pltpu.SemaphoreType.DMA,
        ],
    )
    def k(v0_hbm, v1_hbm, zero_hbm, dst_hbm, o0_hbm, o1_hbm,
          acc, idx_v, rows_v, sem):
        cid = lax.axis_index("c")
        sid = lax.axis_index("s")

        @pl.when(sid == 0)
        def _():
            pltpu.sync_copy(zero_hbm, acc)

        plsc.subcore_barrier()

        def make_body(v_hbm):
            def body(i, carry):
                off = sid * e_per_s + i * _SC_CH
                pltpu.sync_copy(dst_hbm.at[pl.ds(off, _SC_CH)], idx_v)
                pltpu.async_copy(v_hbm.at[pl.ds(off, _SC_CH)], rows_v,
                                 sem).wait()
                pltpu.sync_copy(rows_v, acc.at[idx_v], add=True)
                return carry
            return body

        @pl.when(cid == 0)
        def _():
            lax.fori_loop(0, nch, make_body(v0_hbm), 0)

        @pl.when(cid == 1)
        def _():
            lax.fori_loop(0, nch, make_body(v1_hbm), 0)

        plsc.subcore_barrier()

        @pl.when(cid == 0)
        def _():
            pltpu.sync_copy(acc.at[pl.ds(sid * stripe, stripe)],
                            o0_hbm.at[pl.ds(sid * stripe, stripe)])

        @pl.when(cid == 1)
        def _():
            pltpu.sync_copy(acc.at[pl.ds(sid * stripe, stripe)],
                            o1_hbm.at[pl.ds(sid * stripe, stripe)])

    zero = jnp.zeros((npad, 128), jnp.float32)
    o0, o1 = k(vals0, vals1, zero, dstv)
    return jnp.concatenate([o0, o1], axis=1)

K = 16
BLK = 256
CHUNK = 1024
NEG_INF = float("-inf")


def _leaky(v):
    return jnp.where(v > 0, v, 0.01 * v)


def _top16_merge(run_vals, run_idx, vals, cbase):
    r = run_vals.shape[0]
    cand_v = jnp.concatenate([run_vals, vals], axis=1)
    ncand = cand_v.shape[1]
    pos2 = lax.broadcasted_iota(jnp.int32, (r, ncand), 1)
    pos16 = lax.broadcasted_iota(jnp.int32, (r, K), 1)
    new_v = []
    new_i = []
    for _ in range(K):
        m = jnp.max(cand_v, axis=1, keepdims=True)
        first = jnp.min(jnp.where(cand_v == m, pos2, ncand),
                        axis=1, keepdims=True)
        from_run = jnp.sum(jnp.where(pos16 == first, run_idx, 0),
                           axis=1, keepdims=True)
        picked_i = jnp.where(first < K, from_run, first - K + cbase)
        new_v.append(m)
        new_i.append(picked_i)
        cand_v = jnp.where(pos2 == first, NEG_INF, cand_v)
    return (jnp.concatenate(new_v, axis=1),
            jnp.concatenate(new_i, axis=1))


def _knn_body(clo_ref, cnt_ref, hp_ref, sqr_ref, sqc_ref, rlo_ref, rhi_ref,
              out_ref):
    b = pl.program_id(0)
    h_blk = hp_ref[pl.ds(b * BLK, BLK), :]
    sq_blk = sqr_ref[...]
    rlo = rlo_ref[...]
    rhi = rhi_ref[...]
    rowid = b * BLK + lax.broadcasted_iota(jnp.int32, (BLK, 1), 0)

    run_v0 = jnp.full((BLK, K), NEG_INF, jnp.float32)
    run_i0 = lax.broadcasted_iota(jnp.int32, (BLK, K), 1)

    clo = clo_ref[b]
    cnt = cnt_ref[b]

    def body(j, carry):
        run_v, run_i = carry
        ci = clo + j
        c = ci * CHUNK
        hc = hp_ref[pl.ds(c, CHUNK), :]
        sq_c = sqc_ref[pl.ds(ci, 1), :]
        dot = lax.dot_general(h_blk, hc, (((1,), (1,)), ((), ())),
                              preferred_element_type=jnp.float32)
        d2 = (sq_blk + sq_c) - 2.0 * dot
        colid = c + lax.broadcasted_iota(jnp.int32, (1, CHUNK), 1)
        valid = (colid >= rlo) & (colid < rhi) & (rowid != colid)
        neg = jnp.where(valid, -d2, NEG_INF)
        return _top16_merge(run_v, run_i, neg, c)

    run_v, run_i = lax.fori_loop(0, cnt, body, (run_v0, run_i0))
    out_ref[...] = jnp.pad(run_i, ((0, 0), (0, 128 - K)))


def _knn_pallas(hp, rlo, rhi, clo, cnt):
    npad = hp.shape[0]
    grid = npad // BLK
    f = hp.shape[1]
    nc = npad // CHUNK
    sqv = jnp.sum(hp * hp, axis=1)
    out = pl.pallas_call(
        _knn_body,
        grid_spec=pltpu.PrefetchScalarGridSpec(
            num_scalar_prefetch=2,
            grid=(grid,),
            in_specs=[
                pl.BlockSpec((npad, f), lambda b, *_: (0, 0)),
                pl.BlockSpec((BLK, 1), lambda b, *_: (b, 0)),
                pl.BlockSpec((nc, CHUNK), lambda b, *_: (0, 0)),
                pl.BlockSpec((BLK, 1), lambda b, *_: (b, 0)),
                pl.BlockSpec((BLK, 1), lambda b, *_: (b, 0)),
            ],
            out_specs=pl.BlockSpec((BLK, 128), lambda b, *_: (b, 0)),
        ),
        out_shape=jax.ShapeDtypeStruct((npad, 128), jnp.int32),
    )(clo, cnt, hp, sqv.reshape(npad, 1), sqv.reshape(nc, CHUNK), rlo, rhi)
    return out[:, :K]


def _conv_knn_body(h_ref, hj_ref, wa_ref, ba_ref, wb_ref, bb_ref, out_ref):
    h_blk = h_ref[...]
    hj = hj_ref[...]
    xi = jnp.repeat(h_blk, K, axis=0)
    m = jnp.concatenate([xi, hj - xi], axis=1)
    a1 = _leaky(jnp.dot(m, wa_ref[...], preferred_element_type=jnp.float32)
                + ba_ref[...][None, :])
    a2 = _leaky(jnp.dot(a1, wb_ref[...], preferred_element_type=jnp.float32)
                + bb_ref[...][None, :])
    odim = a2.shape[1]
    a3 = a2.reshape(BLK, K, odim)
    acc = a3[:, 0, :]
    for t in range(1, K):
        acc = acc + a3[:, t, :]
    out_ref[...] = acc


def _conv_knn(h, hj, Wa, ba, Wb, bb):
    npad, f = h.shape
    hdim = Wa.shape[1]
    odim = Wb.shape[1]
    grid = npad // BLK
    return pl.pallas_call(
        _conv_knn_body,
        grid=(grid,),
        in_specs=[
            pl.BlockSpec((BLK, f), lambda b: (b, 0)),
            pl.BlockSpec((BLK * K, f), lambda b: (b, 0)),
            pl.BlockSpec((2 * f, hdim), lambda b: (0, 0)),
            pl.BlockSpec((hdim,), lambda b: (0,)),
            pl.BlockSpec((hdim, odim), lambda b: (0, 0)),
            pl.BlockSpec((odim,), lambda b: (0,)),
        ],
        out_specs=pl.BlockSpec((BLK, odim), lambda b: (b, 0)),
        out_shape=jax.ShapeDtypeStruct((npad, odim), jnp.float32),
    )(h, hj, Wa, ba, Wb, bb)


EBLK = 4096


def _conv1_body(xi_ref, xj_ref, wa_ref, ba_ref, wb_ref, bb_ref, out_ref):
    xi = xi_ref[...]
    m = jnp.concatenate([xi, xj_ref[...] - xi], axis=1)
    a1 = _leaky(jnp.dot(m, wa_ref[...], preferred_element_type=jnp.float32)
                + ba_ref[...][None, :])
    out_ref[...] = _leaky(
        jnp.dot(a1, wb_ref[...], preferred_element_type=jnp.float32)
        + bb_ref[...][None, :])


def _conv1(xi, xj, Wa, ba, Wb, bb):
    epad, f = xi.shape
    hdim = Wa.shape[1]
    odim = Wb.shape[1]
    grid = epad // EBLK
    return pl.pallas_call(
        _conv1_body,
        grid=(grid,),
        in_specs=[
            pl.BlockSpec((EBLK, f), lambda b: (b, 0)),
            pl.BlockSpec((EBLK, f), lambda b: (b, 0)),
            pl.BlockSpec((2 * f, hdim), lambda b: (0, 0)),
            pl.BlockSpec((hdim,), lambda b: (0,)),
            pl.BlockSpec((hdim, odim), lambda b: (0, 0)),
            pl.BlockSpec((odim,), lambda b: (0,)),
        ],
        out_specs=pl.BlockSpec((EBLK, odim), lambda b: (b, 0)),
        out_shape=jax.ShapeDtypeStruct((epad, odim), jnp.float32),
    )(xi, xj, Wa, ba, Wb, bb)


def _head_body(x_ref, s1_ref, s2_ref, s3_ref, s4_ref,
               wpa_ref, bpa_ref, wpb_ref, bpb_ref, wh_ref, bh_ref, o_ref):
    d = x_ref.shape[1]
    acc = jnp.dot(x_ref[...], wpa_ref[:d, :],
                  preferred_element_type=jnp.float32)
    for t, s_ref in enumerate((s1_ref, s2_ref, s3_ref, s4_ref)):
        acc = acc + jnp.dot(s_ref[...], wpa_ref[d + 256 * t:d + 256 * (t + 1), :],
                            preferred_element_type=jnp.float32)
    h1 = _leaky(acc + bpa_ref[...][None, :])
    h2 = _leaky(jnp.dot(h1, wpb_ref[...], preferred_element_type=jnp.float32)
                + bpb_ref[...][None, :])
    o_ref[...] = jnp.dot(h2, wh_ref[...], preferred_element_type=jnp.float32) \
        + bh_ref[...][None, :]


def _head(xp, skips, Wpa, bpa, Wpb, bpb, Wh, bh):
    npad, d = xp.shape
    f = Wpa.shape[0]
    grid = npad // BLK
    out = pl.pallas_call(
        _head_body,
        grid=(grid,),
        in_specs=[
            pl.BlockSpec((BLK, d), lambda i: (i, 0)),
            pl.BlockSpec((BLK, 256), lambda i: (i, 0)),
            pl.BlockSpec((BLK, 256), lambda i: (i, 0)),
            pl.BlockSpec((BLK, 256), lambda i: (i, 0)),
            pl.BlockSpec((BLK, 256), lambda i: (i, 0)),
            pl.BlockSpec((f, 336), lambda i: (0, 0)),
            pl.BlockSpec((336,), lambda i: (0,)),
            pl.BlockSpec((336, 256), lambda i: (0, 0)),
            pl.BlockSpec((256,), lambda i: (0,)),
            pl.BlockSpec((256, 128), lambda i: (0, 0)),
            pl.BlockSpec((128,), lambda i: (0,)),
        ],
        out_specs=pl.BlockSpec((BLK, 128), lambda i: (i, 0)),
        out_shape=jax.ShapeDtypeStruct((npad, 128), jnp.float32),
    )(xp, skips[0], skips[1], skips[2], skips[3], Wpa, bpa, Wpb, bpb,
      jnp.zeros((256, 128), jnp.float32).at[:, :1].set(Wh),
      jnp.zeros((128,), jnp.float32).at[:1].set(bh))
    return out[:, 0]


def kernel(x, W1a, b1a, W1b, b1b, W2a, b2a, W2b, b2b, W3a, b3a, W3b, b3b,
           W4a, b4a, W4b, b4b, Wpa, bpa, Wpb, bpb, Wh, bh, edge_index, batch):
    n, d = x.shape
    npad = ((n + BLK - 1) // BLK) * BLK
    e = edge_index.shape[1]
    epad = ((e + EBLK - 1) // EBLK) * EBLK

    batchp = jnp.full((npad,), 127, jnp.int32).at[:n].set(batch)
    rlo = jnp.searchsorted(batchp, batchp, side="left").astype(jnp.int32)
    rhi = jnp.searchsorted(batchp, batchp, side="right").astype(jnp.int32)
    lo = rlo[::BLK]
    hi = rhi[BLK - 1:: BLK]
    clo = lo // CHUNK
    cnt = (hi + CHUNK - 1) // CHUNK - clo
    rlo = rlo.reshape(npad, 1)
    rhi = rhi.reshape(npad, 1)

    src0 = jnp.zeros((epad,), jnp.int32).at[:e].set(edge_index[0])
    dst0 = jnp.full((epad,), n, jnp.int32).at[:e].set(edge_index[1])
    xp = jnp.zeros((npad, d), jnp.float32).at[:n].set(x)
    xi = _sc_gather(xp, dst0)
    xj = _sc_gather(xp, src0)
    hdn = _conv1(xi, xj, W1a, b1a, W1b, b1b)
    h1 = jax.ops.segment_sum(hdn, dst0, num_segments=n)
    h1p = jnp.zeros((npad, 256), jnp.float32).at[:n].set(h1)

    hp = h1p
    skips = [h1p]
    for (Wa, ba, Wb, bb) in ((W2a, b2a, W2b, b2b), (W3a, b3a, W3b, b3b),
                             (W4a, b4a, W4b, b4b)):
        idx = _knn_pallas(hp, rlo, rhi, clo, cnt)
        hj = _sc_gather(hp, idx.reshape(-1))
        hp = _conv_knn(hp, hj, Wa, ba, Wb, bb)
        skips.append(hp)

    return _head(xp, skips, Wpa, bpa, Wpb, bpb, Wh, bh)[:n]


# speedup vs baseline: 1.1233x; 1.0441x over previous
import functools

import jax
import jax.numpy as jnp
import numpy as np
from jax import lax
from jax.experimental import pallas as pl
from jax.experimental.pallas import tpu as pltpu
from jax.experimental.pallas import tpu_sc as plsc


_SC_NW = 32
_SC_CH = 128


def _sc_gather(table, idx):
    b = idx.shape[0]
    d = table.shape[1]
    b_per_w = b // _SC_NW
    nch = b_per_w // _SC_CH
    nbuf = 4 if d <= 128 else 2
    ngrp = nch // nbuf
    mesh = plsc.VectorSubcoreMesh(core_axis_name="c", subcore_axis_name="s")

    @functools.partial(
        pl.kernel, mesh=mesh,
        out_type=jax.ShapeDtypeStruct((b, d), jnp.float32),
        scratch_types=[
            pltpu.VMEM((b_per_w,), jnp.int32),
            pltpu.VMEM((nbuf, _SC_CH, d), jnp.float32),
            pltpu.SemaphoreType.DMA,
        ],
    )
    def k(table_hbm, idx_hbm, out_hbm, idx_v, rows_v, sem):
        wid = lax.axis_index("s") * 2 + lax.axis_index("c")
        base = wid * b_per_w
        pltpu.sync_copy(idx_hbm.at[pl.ds(base, b_per_w)], idx_v)

        def body(g, carry):
            off0 = g * (nbuf * _SC_CH)
            copies = []
            for u in range(nbuf):
                copies.append(pltpu.async_copy(
                    table_hbm.at[idx_v.at[pl.ds(off0 + u * _SC_CH, _SC_CH)]],
                    rows_v.at[u], sem))
            for u in range(nbuf):
                copies[u].wait()
                pltpu.sync_copy(
                    rows_v.at[u],
                    out_hbm.at[pl.ds(base + off0 + u * _SC_CH, _SC_CH)])
            return carry

        lax.fori_loop(0, ngrp, body, 0)

    return k(table, idx)


def _sc_scatter_add(vals0, vals1, dstv, npad):
    epad = dstv.shape[0]
    e_per_s = epad // 16
    nch = e_per_s // _SC_CH
    stripe = npad // 16
    mesh = plsc.VectorSubcoreMesh(core_axis_name="c", subcore_axis_name="s")

    @functools.partial(
        pl.kernel, mesh=mesh,
        out_type=(jax.ShapeDtypeStruct((npad, 128), jnp.float32),
                  jax.ShapeDtypeStruct((npad, 128), jnp.float32)),
        scratch_types=[
            pltpu.VMEM_SHARED((npad, 128), jnp.float32),
            pltpu.VMEM((_SC_CH,), jnp.int32),
            pltpu.VMEM((_SC_CH, 128), jnp.float32),
            pltpu.SemaphoreType.DMA,
        ],
    )
    def k(v0_hbm, v1_hbm, zero_hbm, dst_hbm, o0_hbm, o1_hbm,
          acc, idx_v, rows_v, sem):
        cid = lax.axis_index("c")
        sid = lax.axis_index("s")

        @pl.when(sid == 0)
        def _():
            pltpu.sync_copy(zero_hbm, acc)

        plsc.subcore_barrier()

        def make_body(v_hbm):
            def body(i, carry):
                off = sid * e_per_s + i * _SC_CH
                pltpu.sync_copy(dst_hbm.at[pl.ds(off, _SC_CH)], idx_v)
                pltpu.async_copy(v_hbm.at[pl.ds(off, _SC_CH)], rows_v,
                                 sem).wait()
                pltpu.sync_copy(rows_v, acc.at[idx_v], add=True)
                return carry
            return body

        @pl.when(cid == 0)
        def _():
            lax.fori_loop(0, nch, make_body(v0_hbm), 0)

        @pl.when(cid == 1)
        def _():
            lax.fori_loop(0, nch, make_body(v1_hbm), 0)

        plsc.subcore_barrier()

        @pl.when(cid == 0)
        def _():
            pltpu.sync_copy(acc.at[pl.ds(sid * stripe, stripe)],
                            o0_hbm.at[pl.ds(sid * stripe, stripe)])

        @pl.when(cid == 1)
        def _():
            pltpu.sync_copy(acc.at[pl.ds(sid * stripe, stripe)],
                            o1_hbm.at[pl.ds(sid * stripe, stripe)])

    zero = jnp.zeros((npad, 128), jnp.float32)
    o0, o1 = k(vals0, vals1, zero, dstv)
    return jnp.concatenate([o0, o1], axis=1)

K = 16
BLK = 256
CHUNK = 1024
NEG_INF = float("-inf")


def _leaky(v):
    return jnp.where(v > 0, v, 0.01 * v)


def _top16_merge(run_vals, run_idx, vals, cbase):
    r = run_vals.shape[0]
    cand_v = jnp.concatenate([run_vals, vals], axis=1)
    ncand = cand_v.shape[1]
    pos2 = lax.broadcasted_iota(jnp.int32, (r, ncand), 1)
    pos16 = lax.broadcasted_iota(jnp.int32, (r, K), 1)
    new_v = []
    new_i = []
    for _ in range(K):
        m = jnp.max(cand_v, axis=1, keepdims=True)
        first = jnp.min(jnp.where(cand_v == m, pos2, ncand),
                        axis=1, keepdims=True)
        from_run = jnp.sum(jnp.where(pos16 == first, run_idx, 0),
                           axis=1, keepdims=True)
        picked_i = jnp.where(first < K, from_run, first - K + cbase)
        new_v.append(m)
        new_i.append(picked_i)
        cand_v = jnp.where(pos2 == first, NEG_INF, cand_v)
    return (jnp.concatenate(new_v, axis=1),
            jnp.concatenate(new_i, axis=1))


def _knn_body(clo_ref, cnt_ref, hp_ref, sqr_ref, sqc_ref, rlo_ref, rhi_ref,
              out_ref):
    b = pl.program_id(0)
    h_blk = hp_ref[pl.ds(b * BLK, BLK), :]
    sq_blk = sqr_ref[...]
    rlo = rlo_ref[...]
    rhi = rhi_ref[...]
    rowid = b * BLK + lax.broadcasted_iota(jnp.int32, (BLK, 1), 0)

    run_v0 = jnp.full((BLK, K), NEG_INF, jnp.float32)
    run_i0 = lax.broadcasted_iota(jnp.int32, (BLK, K), 1)

    clo = clo_ref[b]
    cnt = cnt_ref[b]

    def body(j, carry):
        run_v, run_i = carry
        ci = clo + j
        c = ci * CHUNK
        hc = hp_ref[pl.ds(c, CHUNK), :]
        sq_c = sqc_ref[pl.ds(ci, 1), :]
        dot = lax.dot_general(h_blk, hc, (((1,), (1,)), ((), ())),
                              preferred_element_type=jnp.float32)
        d2 = (sq_blk + sq_c) - 2.0 * dot
        colid = c + lax.broadcasted_iota(jnp.int32, (1, CHUNK), 1)
        valid = (colid >= rlo) & (colid < rhi) & (rowid != colid)
        neg = jnp.where(valid, -d2, NEG_INF)
        return _top16_merge(run_v, run_i, neg, c)

    run_v, run_i = lax.fori_loop(0, cnt, body, (run_v0, run_i0))
    out_ref[...] = jnp.pad(run_i, ((0, 0), (0, 128 - K)))


def _knn_pallas(hp, rlo, rhi, clo, cnt):
    npad = hp.shape[0]
    grid = npad // BLK
    f = hp.shape[1]
    nc = npad // CHUNK
    sqv = jnp.sum(hp * hp, axis=1)
    out = pl.pallas_call(
        _knn_body,
        grid_spec=pltpu.PrefetchScalarGridSpec(
            num_scalar_prefetch=2,
            grid=(grid,),
            in_specs=[
                pl.BlockSpec((npad, f), lambda b, *_: (0, 0)),
                pl.BlockSpec((BLK, 1), lambda b, *_: (b, 0)),
                pl.BlockSpec((nc, CHUNK), lambda b, *_: (0, 0)),
                pl.BlockSpec((BLK, 1), lambda b, *_: (b, 0)),
                pl.BlockSpec((BLK, 1), lambda b, *_: (b, 0)),
            ],
            out_specs=pl.BlockSpec((BLK, 128), lambda b, *_: (b, 0)),
        ),
        out_shape=jax.ShapeDtypeStruct((npad, 128), jnp.int32),
    )(clo, cnt, hp, sqv.reshape(npad, 1), sqv.reshape(nc, CHUNK), rlo, rhi)
    return out[:, :K]


def _conv_knn_body(h_ref, hj_ref, wa_ref, ba_ref, wb_ref, bb_ref, out_ref):
    h_blk = h_ref[...]
    hj = hj_ref[...]
    xi = jnp.repeat(h_blk, K, axis=0)
    m = jnp.concatenate([xi, hj - xi], axis=1)
    a1 = _leaky(jnp.dot(m, wa_ref[...], preferred_element_type=jnp.float32)
                + ba_ref[...][None, :])
    a2 = _leaky(jnp.dot(a1, wb_ref[...], preferred_element_type=jnp.float32)
                + bb_ref[...][None, :])
    odim = a2.shape[1]
    a3 = a2.reshape(BLK, K, odim)
    acc = a3[:, 0, :]
    for t in range(1, K):
        acc = acc + a3[:, t, :]
    out_ref[...] = acc


def _conv_knn(h, hj, Wa, ba, Wb, bb):
    npad, f = h.shape
    hdim = Wa.shape[1]
    odim = Wb.shape[1]
    grid = npad // BLK
    return pl.pallas_call(
        _conv_knn_body,
        grid=(grid,),
        in_specs=[
            pl.BlockSpec((BLK, f), lambda b: (b, 0)),
            pl.BlockSpec((BLK * K, f), lambda b: (b, 0)),
            pl.BlockSpec((2 * f, hdim), lambda b: (0, 0)),
            pl.BlockSpec((hdim,), lambda b: (0,)),
            pl.BlockSpec((hdim, odim), lambda b: (0, 0)),
            pl.BlockSpec((odim,), lambda b: (0,)),
        ],
        out_specs=pl.BlockSpec((BLK, odim), lambda b: (b, 0)),
        out_shape=jax.ShapeDtypeStruct((npad, odim), jnp.float32),
    )(h, hj, Wa, ba, Wb, bb)


EBLK = 4096


def _conv1_body(xi_ref, xj_ref, wa_ref, ba_ref, wb_ref, bb_ref, out_ref):
    xi = xi_ref[...]
    m = jnp.concatenate([xi, xj_ref[...] - xi], axis=1)
    a1 = _leaky(jnp.dot(m, wa_ref[...], preferred_element_type=jnp.float32)
                + ba_ref[...][None, :])
    out_ref[...] = _leaky(
        jnp.dot(a1, wb_ref[...], preferred_element_type=jnp.float32)
        + bb_ref[...][None, :])


def _conv1(xi, xj, Wa, ba, Wb, bb):
    epad, f = xi.shape
    hdim = Wa.shape[1]
    odim = Wb.shape[1]
    grid = epad // EBLK
    return pl.pallas_call(
        _conv1_body,
        grid=(grid,),
        in_specs=[
            pl.BlockSpec((EBLK, f), lambda b: (b, 0)),
            pl.BlockSpec((EBLK, f), lambda b: (b, 0)),
            pl.BlockSpec((2 * f, hdim), lambda b: (0, 0)),
            pl.BlockSpec((hdim,), lambda b: (0,)),
            pl.BlockSpec((hdim, odim), lambda b: (0, 0)),
            pl.BlockSpec((odim,), lambda b: (0,)),
        ],
        out_specs=pl.BlockSpec((EBLK, odim), lambda b: (b, 0)),
        out_shape=jax.ShapeDtypeStruct((epad, odim), jnp.float32),
    )(xi, xj, Wa, ba, Wb, bb)


def _head_body(x_ref, s1_ref, s2_ref, s3_ref, s4_ref,
               wpa_ref, bpa_ref, wpb_ref, bpb_ref, wh_ref, bh_ref, o_ref):
    d = x_ref.shape[1]
    acc = jnp.dot(x_ref[...], wpa_ref[:d, :],
                  preferred_element_type=jnp.float32)
    for t, s_ref in enumerate((s1_ref, s2_ref, s3_ref, s4_ref)):
        acc = acc + jnp.dot(s_ref[...], wpa_ref[d + 256 * t:d + 256 * (t + 1), :],
                            preferred_element_type=jnp.float32)
    h1 = _leaky(acc + bpa_ref[...][None, :])
    h2 = _leaky(jnp.dot(h1, wpb_ref[...], preferred_element_type=jnp.float32)
                + bpb_ref[...][None, :])
    o_ref[...] = jnp.dot(h2, wh_ref[...], preferred_element_type=jnp.float32) \
        + bh_ref[...][None, :]


def _head(xp, skips, Wpa, bpa, Wpb, bpb, Wh, bh):
    npad, d = xp.shape
    f = Wpa.shape[0]
    grid = npad // BLK
    out = pl.pallas_call(
        _head_body,
        grid=(grid,),
        in_specs=[
            pl.BlockSpec((BLK, d), lambda i: (i, 0)),
            pl.BlockSpec((BLK, 256), lambda i: (i, 0)),
            pl.BlockSpec((BLK, 256), lambda i: (i, 0)),
            pl.BlockSpec((BLK, 256), lambda i: (i, 0)),
            pl.BlockSpec((BLK, 256), lambda i: (i, 0)),
            pl.BlockSpec((f, 336), lambda i: (0, 0)),
            pl.BlockSpec((336,), lambda i: (0,)),
            pl.BlockSpec((336, 256), lambda i: (0, 0)),
            pl.BlockSpec((256,), lambda i: (0,)),
            pl.BlockSpec((256, 128), lambda i: (0, 0)),
            pl.BlockSpec((128,), lambda i: (0,)),
        ],
        out_specs=pl.BlockSpec((BLK, 128), lambda i: (i, 0)),
        out_shape=jax.ShapeDtypeStruct((npad, 128), jnp.float32),
    )(xp, skips[0], skips[1], skips[2], skips[3], Wpa, bpa, Wpb, bpb,
      jnp.zeros((256, 128), jnp.float32).at[:, :1].set(Wh),
      jnp.zeros((128,), jnp.float32).at[:1].set(bh))
    return out[:, 0]


def kernel(x, W1a, b1a, W1b, b1b, W2a, b2a, W2b, b2b, W3a, b3a, W3b, b3b,
           W4a, b4a, W4b, b4b, Wpa, bpa, Wpb, bpb, Wh, bh, edge_index, batch):
    n, d = x.shape
    npad = ((n + BLK - 1) // BLK) * BLK
    e = edge_index.shape[1]
    epad = ((e + EBLK - 1) // EBLK) * EBLK

    batchp = jnp.full((npad,), 127, jnp.int32).at[:n].set(batch)
    rlo = jnp.searchsorted(batchp, batchp, side="left").astype(jnp.int32)
    rhi = jnp.searchsorted(batchp, batchp, side="right").astype(jnp.int32)
    lo = rlo[::BLK]
    hi = rhi[BLK - 1:: BLK]
    clo = lo // CHUNK
    cnt = (hi + CHUNK - 1) // CHUNK - clo
    rlo = rlo.reshape(npad, 1)
    rhi = rhi.reshape(npad, 1)

    src0 = jnp.zeros((epad,), jnp.int32).at[:e].set(edge_index[0])
    dst0 = jnp.full((epad,), n, jnp.int32).at[:e].set(edge_index[1])
    xp = jnp.zeros((npad, d), jnp.float32).at[:n].set(x)
    xi = _sc_gather(xp, dst0)
    xj = _sc_gather(xp, src0)
    hdn = _conv1(xi, xj, W1a, b1a, W1b, b1b)
    h1 = jax.ops.segment_sum(hdn, dst0, num_segments=n)
    h1p = jnp.zeros((npad, 256), jnp.float32).at[:n].set(h1)

    hp = h1p
    skips = [h1p]
    for (Wa, ba, Wb, bb) in ((W2a, b2a, W2b, b2b), (W3a, b3a, W3b, b3b),
                             (W4a, b4a, W4b, b4b)):
        idx = _knn_pallas(hp, rlo, rhi, clo, cnt)
        hj = _sc_gather(hp, idx.reshape(-1))
        hp = _conv_knn(hp, hj, Wa, ba, Wb, bb)
        skips.append(hp)

    return _head(xp, skips, Wpa, bpa, Wpb, bpb, Wh, bh)[:n]
